# Initial kernel scaffold; baseline (speedup 1.0000x reference)
import jax
import jax.numpy as jnp
from jax.experimental import pallas as pl


def kernel(inputs, weight):
    raise NotImplementedError("write your pallas kernel here")


# speedup vs baseline: 1.4761x; 1.4761x over previous
import functools

import jax
import jax.numpy as jnp
from jax import lax
from jax.experimental import pallas as pl
from jax.experimental.pallas import tpu as pltpu
from jax.experimental.pallas import tpu_sc as plsc

EPS = 1e-05
L = 16
NC = 2
NS = 16
NW = NC * NS


def _sc_gather_reduce(weight, idx_flat, B, S, D):
    bpw = B // NW
    C = 32
    nchunk = bpw // C
    rows_per_chunk = C * S

    def body(weight_hbm, idx_hbm, nv_hbm, sq_hbm,
             idx_v, rows_v, onv_v, osq_v, gsem):
        wid = lax.axis_index("s") * NC + lax.axis_index("c")
        iota = lax.iota(jnp.int32, L)
        zeros_f = jnp.zeros((L,), jnp.float32)

        def chunk_body(k, _):
            base_e = (wid * bpw + k * C) * S
            pltpu.sync_copy(idx_hbm.at[pl.ds(base_e, rows_per_chunk)], idx_v)
            pltpu.async_copy(weight_hbm.at[idx_v], rows_v, gsem).wait()
            for g in range(C // L):
                rbase = (g * L + iota) * S
                u = [plsc.load_gather(rows_v,
                                      [rbase, jnp.full((L,), d, jnp.int32)])
                     for d in range(D)]
                nu = u[0] * u[0]
                for d in range(1, D):
                    nu = nu + u[d] * u[d]
                plsc.store_scatter(onv_v, [rbase], nu)
                plsc.store_scatter(osq_v, [rbase], zeros_f)

                def s_body(s, _):
                    ridx = rbase + s
                    acc_sq = zeros_f
                    acc_nv = zeros_f
                    for d in range(D):
                        vd = plsc.load_gather(
                            rows_v, [ridx, jnp.full((L,), d, jnp.int32)])
                        diff = u[d] - vd
                        acc_sq = acc_sq + diff * diff
                        acc_nv = acc_nv + vd * vd
                    plsc.store_scatter(onv_v, [ridx], acc_nv)
                    plsc.store_scatter(osq_v, [ridx], acc_sq)
                    return 0

                lax.fori_loop(1, S, s_body, 0)
            pltpu.sync_copy(onv_v, nv_hbm.at[pl.ds(base_e, rows_per_chunk)])
            pltpu.sync_copy(osq_v, sq_hbm.at[pl.ds(base_e, rows_per_chunk)])
            return 0

        lax.fori_loop(0, nchunk, chunk_body, 0)

    mesh = plsc.VectorSubcoreMesh(core_axis_name="c", subcore_axis_name="s")
    f = pl.kernel(
        body,
        out_type=[jax.ShapeDtypeStruct((B * S,), jnp.float32),
                  jax.ShapeDtypeStruct((B * S,), jnp.float32)],
        mesh=mesh,
        compiler_params=pltpu.CompilerParams(needs_layout_passes=False, use_tc_tiling_on_sc=False),
        scratch_types=[
            pltpu.VMEM((rows_per_chunk,), jnp.int32),
            pltpu.VMEM((rows_per_chunk, D), jnp.float32),
            pltpu.VMEM((rows_per_chunk,), jnp.float32),
            pltpu.VMEM((rows_per_chunk,), jnp.float32),
            pltpu.SemaphoreType.DMA,
        ],
    )
    return f(weight, idx_flat)


def _tc_finish_body(nv_ref, sq_ref, out_ref):
    nv = nv_ref[...]
    sq = sq_ref[...]
    nu = nv[:, 0:1]
    su = jnp.where(nu > 1.0, 1.0 / (jnp.sqrt(nu) + 1e-7), 1.0)
    sv = jnp.where(nv > 1.0, 1.0 / (jnp.sqrt(nv) + 1e-7), 1.0)
    nu_c = jnp.clip(su * su * nu, 0.0, 1.0 - EPS)
    nv_c = jnp.clip(sv * sv * nv, 0.0, 1.0 - EPS)
    sq_s = su * su * nu + sv * sv * nv - su * sv * (nu + nv - sq)
    x = 1.0 + 2.0 * sq_s / ((1.0 - nu_c) * (1.0 - nv_c))
    x = jnp.maximum(x, 1.0 + EPS)
    dist = jnp.log(x + jnp.sqrt((x - 1.0) * (x + 1.0)))
    out_ref[...] = dist[:, 1:]


def _tc_finish(nv, sq, B, S):
    blk = 512
    return pl.pallas_call(
        _tc_finish_body,
        grid=(B // blk,),
        in_specs=[pl.BlockSpec((blk, S), lambda i: (i, 0)),
                  pl.BlockSpec((blk, S), lambda i: (i, 0))],
        out_specs=pl.BlockSpec((blk, S - 1), lambda i: (i, 0)),
        out_shape=jax.ShapeDtypeStruct((B, S - 1), jnp.float32),
    )(nv, sq)


@jax.jit
def kernel(inputs, weight):
    B, S = inputs.shape
    _, D = weight.shape
    idx_flat = inputs.reshape(B * S)
    nv_flat, sq_flat = _sc_gather_reduce(weight, idx_flat, B, S, D)
    nv = nv_flat.reshape(B, S)
    sq = sq_flat.reshape(B, S)
    return _tc_finish(nv, sq, B, S)


# speedup vs baseline: 1.9055x; 1.2909x over previous
import functools

import jax
import jax.numpy as jnp
from jax import lax
from jax.experimental import pallas as pl
from jax.experimental.pallas import tpu as pltpu
from jax.experimental.pallas import tpu_sc as plsc

EPS = 1e-05
L = 16
NC = 2
NS = 16
NW = NC * NS


def _sc_gather_reduce(weight, idx_flat, B, S, D):
    bpw = B // NW
    C = 32
    nchunk = bpw // C
    rows_per_chunk = C * S

    def body(weight_hbm, idx_hbm, nv_hbm, sq_hbm,
             idx_v0, idx_v1, rows_v0, rows_v1, onv_v0, onv_v1,
             osq_v0, osq_v1, gsem0, gsem1, osem0, osem1):
        idx_v = (idx_v0, idx_v1)
        rows_v = (rows_v0, rows_v1)
        onv_v = (onv_v0, onv_v1)
        osq_v = (osq_v0, osq_v1)
        gsem = (gsem0, gsem1)
        osem = (osem0, osem1)
        wid = lax.axis_index("s") * NC + lax.axis_index("c")
        iota = lax.iota(jnp.int32, L)
        zeros_f = jnp.zeros((L,), jnp.float32)
        cols = [(jnp.full((L,), d, jnp.int32) + iota) & (D - 1)
                for d in range(D)]
        rbases = [(g * L + iota) * S for g in range(C // L)]

        def start_fetch(k, b):
            base_e = (wid * bpw + k * C) * S
            pltpu.sync_copy(idx_hbm.at[pl.ds(base_e, rows_per_chunk)],
                            idx_v[b])
            return pltpu.async_copy(weight_hbm.at[idx_v[b]],
                                    rows_v[b], gsem[b])

        def compute(b):
            for g in range(C // L):
                rbase = rbases[g]
                u = [plsc.load_gather(rows_v[b], [rbase, cols[d]])
                     for d in range(D)]
                nu = u[0] * u[0]
                for d in range(1, D):
                    nu = nu + u[d] * u[d]
                plsc.store_scatter(onv_v[b], [rbase], nu)
                plsc.store_scatter(osq_v[b], [rbase], zeros_f)

                def s_body(s, _):
                    ridx = rbase + s
                    acc_sq = zeros_f
                    acc_nv = zeros_f
                    for d in range(D):
                        vd = plsc.load_gather(rows_v[b], [ridx, cols[d]])
                        diff = u[d] - vd
                        acc_sq = acc_sq + diff * diff
                        acc_nv = acc_nv + vd * vd
                    plsc.store_scatter(onv_v[b], [ridx], acc_nv)
                    plsc.store_scatter(osq_v[b], [ridx], acc_sq)
                    return 0

                lax.fori_loop(1, S, s_body, 0)

        def start_flush(k, b):
            base_e = (wid * bpw + k * C) * S
            return (pltpu.async_copy(onv_v[b],
                                     nv_hbm.at[pl.ds(base_e, rows_per_chunk)],
                                     osem[b]),
                    pltpu.async_copy(osq_v[b],
                                     sq_hbm.at[pl.ds(base_e, rows_per_chunk)],
                                     osem[b]))

        gather = {0: start_fetch(0, 0)}
        flush = {}
        for k in range(nchunk):
            b = k % 2
            if k + 1 < nchunk:
                gather[k + 1] = start_fetch(k + 1, (k + 1) % 2)
            gather.pop(k).wait()
            if k - 2 in flush:
                for h in flush.pop(k - 2):
                    h.wait()
            compute(b)
            flush[k] = start_flush(k, b)
        for hs in flush.values():
            for h in hs:
                h.wait()

    mesh = plsc.VectorSubcoreMesh(core_axis_name="c", subcore_axis_name="s")
    f = pl.kernel(
        body,
        out_type=[jax.ShapeDtypeStruct((B * S,), jnp.float32),
                  jax.ShapeDtypeStruct((B * S,), jnp.float32)],
        mesh=mesh,
        compiler_params=pltpu.CompilerParams(needs_layout_passes=False,
                                             use_tc_tiling_on_sc=False),
        scratch_types=[
            pltpu.VMEM((rows_per_chunk,), jnp.int32),
            pltpu.VMEM((rows_per_chunk,), jnp.int32),
            pltpu.VMEM((rows_per_chunk, D), jnp.float32),
            pltpu.VMEM((rows_per_chunk, D), jnp.float32),
            pltpu.VMEM((rows_per_chunk,), jnp.float32),
            pltpu.VMEM((rows_per_chunk,), jnp.float32),
            pltpu.VMEM((rows_per_chunk,), jnp.float32),
            pltpu.VMEM((rows_per_chunk,), jnp.float32),
            pltpu.SemaphoreType.DMA,
            pltpu.SemaphoreType.DMA,
            pltpu.SemaphoreType.DMA,
            pltpu.SemaphoreType.DMA,
        ],
    )
    return f(weight, idx_flat)


def _tc_finish_body(nv_ref, sq_ref, out_ref):
    nv = nv_ref[...]
    sq = sq_ref[...]
    nu = nv[:, 0:1]
    su = jnp.where(nu > 1.0, 1.0 / (jnp.sqrt(nu) + 1e-7), 1.0)
    sv = jnp.where(nv > 1.0, 1.0 / (jnp.sqrt(nv) + 1e-7), 1.0)
    nu_c = jnp.clip(su * su * nu, 0.0, 1.0 - EPS)
    nv_c = jnp.clip(sv * sv * nv, 0.0, 1.0 - EPS)
    sq_s = su * su * nu + sv * sv * nv - su * sv * (nu + nv - sq)
    x = 1.0 + 2.0 * sq_s / ((1.0 - nu_c) * (1.0 - nv_c))
    x = jnp.maximum(x, 1.0 + EPS)
    dist = jnp.log(x + jnp.sqrt((x - 1.0) * (x + 1.0)))
    out_ref[...] = dist[:, 1:]


def _tc_finish(nv, sq, B, S):
    blk = 512
    return pl.pallas_call(
        _tc_finish_body,
        grid=(B // blk,),
        in_specs=[pl.BlockSpec((blk, S), lambda i: (i, 0)),
                  pl.BlockSpec((blk, S), lambda i: (i, 0))],
        out_specs=pl.BlockSpec((blk, S - 1), lambda i: (i, 0)),
        out_shape=jax.ShapeDtypeStruct((B, S - 1), jnp.float32),
    )(nv, sq)


@jax.jit
def kernel(inputs, weight):
    B, S = inputs.shape
    _, D = weight.shape
    idx_flat = inputs.reshape(B * S)
    nv_flat, sq_flat = _sc_gather_reduce(weight, idx_flat, B, S, D)
    nv = nv_flat.reshape(B, S)
    sq = sq_flat.reshape(B, S)
    return _tc_finish(nv, sq, B, S)


# speedup vs baseline: 2.3828x; 1.2505x over previous
import functools

import jax
import jax.numpy as jnp
from jax import lax
from jax.experimental import pallas as pl
from jax.experimental.pallas import tpu as pltpu
from jax.experimental.pallas import tpu_sc as plsc

EPS = 1e-05
L = 16
NC = 2
NS = 16
NW = NC * NS


def _sc_gather_reduce(weight, idx_flat, B, S, D):
    bpw = B // NW
    C = 32
    nchunk = bpw // C
    rows_per_chunk = C * S

    def body(weight_hbm, idx_hbm, nv_hbm, sq_hbm,
             idx_v0, idx_v1, rows_v0, rows_v1, onv_v0, onv_v1,
             osq_v0, osq_v1, gsem0, gsem1, osem0, osem1):
        idx_v = (idx_v0, idx_v1)
        rows_v = (rows_v0, rows_v1)
        onv_v = (onv_v0, onv_v1)
        osq_v = (osq_v0, osq_v1)
        gsem = (gsem0, gsem1)
        osem = (osem0, osem1)
        wid = lax.axis_index("s") * NC + lax.axis_index("c")
        iota = lax.iota(jnp.int32, L)
        zeros_f = jnp.zeros((L,), jnp.float32)
        cols = [(jnp.full((L,), d, jnp.int32) + iota) & (D - 1)
                for d in range(D)]
        rbases = [(g * L + iota) * S for g in range(C // L)]

        def start_fetch(k, b):
            base_e = (wid * bpw + k * C) * S
            pltpu.sync_copy(idx_hbm.at[pl.ds(base_e, rows_per_chunk)],
                            idx_v[b])
            return pltpu.async_copy(weight_hbm.at[idx_v[b]],
                                    rows_v[b], gsem[b])

        def compute(b):
            for g in range(C // L):
                rbase = rbases[g]
                u = [plsc.load_gather(rows_v[b], [rbase, cols[d]])
                     for d in range(D)]
                nu = u[0] * u[0]
                for d in range(1, D):
                    nu = nu + u[d] * u[d]
                plsc.store_scatter(onv_v[b], [rbase], nu)
                plsc.store_scatter(osq_v[b], [rbase], zeros_f)

                def s_body(s, _):
                    ridx = rbase + s
                    acc_sq = zeros_f
                    acc_nv = zeros_f
                    for d in range(D):
                        vd = plsc.load_gather(rows_v[b], [ridx, cols[d]])
                        diff = u[d] - vd
                        acc_sq = acc_sq + diff * diff
                        acc_nv = acc_nv + vd * vd
                    plsc.store_scatter(onv_v[b], [ridx], acc_nv)
                    plsc.store_scatter(osq_v[b], [ridx], acc_sq)
                    return 0

                lax.fori_loop(1, S, s_body, 0)

        def start_flush(k, b):
            base_e = (wid * bpw + k * C) * S
            return (pltpu.async_copy(onv_v[b],
                                     nv_hbm.at[pl.ds(base_e, rows_per_chunk)],
                                     osem[b]),
                    pltpu.async_copy(osq_v[b],
                                     sq_hbm.at[pl.ds(base_e, rows_per_chunk)],
                                     osem[b]))

        gather = {0: start_fetch(0, 0)}
        flush = {}
        for k in range(nchunk):
            b = k % 2
            if k + 1 < nchunk:
                gather[k + 1] = start_fetch(k + 1, (k + 1) % 2)
            gather.pop(k).wait()
            if k - 2 in flush:
                for h in flush.pop(k - 2):
                    h.wait()
            compute(b)
            flush[k] = start_flush(k, b)
        for hs in flush.values():
            for h in hs:
                h.wait()

    mesh = plsc.VectorSubcoreMesh(core_axis_name="c", subcore_axis_name="s")
    f = pl.kernel(
        body,
        out_type=[jax.ShapeDtypeStruct((B * S,), jnp.float32),
                  jax.ShapeDtypeStruct((B * S,), jnp.float32)],
        mesh=mesh,
        compiler_params=pltpu.CompilerParams(needs_layout_passes=False,
                                             use_tc_tiling_on_sc=False),
        scratch_types=[
            pltpu.VMEM((rows_per_chunk,), jnp.int32),
            pltpu.VMEM((rows_per_chunk,), jnp.int32),
            pltpu.VMEM((rows_per_chunk, D), jnp.float32),
            pltpu.VMEM((rows_per_chunk, D), jnp.float32),
            pltpu.VMEM((rows_per_chunk,), jnp.float32),
            pltpu.VMEM((rows_per_chunk,), jnp.float32),
            pltpu.VMEM((rows_per_chunk,), jnp.float32),
            pltpu.VMEM((rows_per_chunk,), jnp.float32),
            pltpu.SemaphoreType.DMA,
            pltpu.SemaphoreType.DMA,
            pltpu.SemaphoreType.DMA,
            pltpu.SemaphoreType.DMA,
        ],
    )
    return f(weight, idx_flat)


def _tc_transpose_body(wt_ref, out_ref):
    y = wt_ref[...].T
    y3 = y.reshape(y.shape[0] // 8, 8, y.shape[1])
    out_ref[...] = jnp.concatenate([y3[:, k, :] for k in range(8)], axis=1)


def _tc_transpose(wt, N, D):
    CT = 4096
    R = N * D // 128
    RB = CT * D // 128
    return pl.pallas_call(
        _tc_transpose_body,
        grid=(pl.cdiv(N, CT),),
        in_specs=[pl.BlockSpec((D, CT), lambda i: (0, i))],
        out_specs=pl.BlockSpec((RB, 128), lambda i: (i, 0)),
        out_shape=jax.ShapeDtypeStruct((R, 128), jnp.float32),
    )(wt)


def _tc_finish_body(nv_ref, sq_ref, out_ref):
    nv = nv_ref[...]
    sq = sq_ref[...]
    nu = nv[:, 0:1]
    su = jnp.where(nu > 1.0, 1.0 / (jnp.sqrt(nu) + 1e-7), 1.0)
    sv = jnp.where(nv > 1.0, 1.0 / (jnp.sqrt(nv) + 1e-7), 1.0)
    nu_c = jnp.clip(su * su * nu, 0.0, 1.0 - EPS)
    nv_c = jnp.clip(sv * sv * nv, 0.0, 1.0 - EPS)
    sq_s = su * su * nu + sv * sv * nv - su * sv * (nu + nv - sq)
    x = 1.0 + 2.0 * sq_s / ((1.0 - nu_c) * (1.0 - nv_c))
    x = jnp.maximum(x, 1.0 + EPS)
    dist = jnp.log(x + jnp.sqrt((x - 1.0) * (x + 1.0)))
    out_ref[...] = dist[:, 1:]


def _tc_finish(nv, sq, B, S):
    blk = 512
    return pl.pallas_call(
        _tc_finish_body,
        grid=(B // blk,),
        in_specs=[pl.BlockSpec((blk, S), lambda i: (i, 0)),
                  pl.BlockSpec((blk, S), lambda i: (i, 0))],
        out_specs=pl.BlockSpec((blk, S - 1), lambda i: (i, 0)),
        out_shape=jax.ShapeDtypeStruct((B, S - 1), jnp.float32),
    )(nv, sq)


@jax.jit
def kernel(inputs, weight):
    B, S = inputs.shape
    N, D = weight.shape
    w_lin = _tc_transpose(weight.T, N, D)
    idx_flat = inputs.reshape(B * S)
    nv_flat, sq_flat = _sc_gather_reduce(w_lin.reshape(N, D), idx_flat,
                                         B, S, D)
    nv = nv_flat.reshape(B, S)
    sq = sq_flat.reshape(B, S)
    return _tc_finish(nv, sq, B, S)


# speedup vs baseline: 2.5383x; 1.0653x over previous
import functools

import jax
import jax.numpy as jnp
from jax import lax
from jax.experimental import pallas as pl
from jax.experimental.pallas import tpu as pltpu
from jax.experimental.pallas import tpu_sc as plsc

EPS = 1e-05
L = 16
NC = 2
NS = 16
NW = NC * NS


def _sc_gather_reduce(weight, idx_flat, B, S, D):
    bpw = B // NW
    C = 32
    nchunk = bpw // C
    rows_per_chunk = C * S

    def body(weight_hbm, idx_hbm, nv_hbm, sq_hbm,
             idx_v0, idx_v1, rows_v0, rows_v1, onv_v0, onv_v1,
             osq_v0, osq_v1, gsem0, gsem1, osem0, osem1):
        idx_v = (idx_v0, idx_v1)
        rows_v = (rows_v0, rows_v1)
        onv_v = (onv_v0, onv_v1)
        osq_v = (osq_v0, osq_v1)
        gsem = (gsem0, gsem1)
        osem = (osem0, osem1)
        wid = lax.axis_index("s") * NC + lax.axis_index("c")
        iota = lax.iota(jnp.int32, L)
        zeros_f = jnp.zeros((L,), jnp.float32)
        cols = [(jnp.full((L,), d, jnp.int32) + iota) & (D - 1)
                for d in range(D)]
        rbases = [(g * L + iota) * S for g in range(C // L)]

        def start_fetch(k, b):
            base_e = (wid * bpw + k * C) * S
            pltpu.sync_copy(idx_hbm.at[pl.ds(base_e, rows_per_chunk)],
                            idx_v[b])
            return pltpu.async_copy(weight_hbm.at[idx_v[b]],
                                    rows_v[b], gsem[b])

        def compute(b):
            for g in range(C // L):
                rbase = rbases[g]
                u = [plsc.load_gather(rows_v[b], [rbase, cols[d]])
                     for d in range(D)]
                nu = u[0] * u[0]
                for d in range(1, D):
                    nu = nu + u[d] * u[d]
                plsc.store_scatter(onv_v[b], [rbase], nu)
                plsc.store_scatter(osq_v[b], [rbase], zeros_f)

                def s_body(s, _):
                    ridx = rbase + s
                    acc_sq = zeros_f
                    acc_nv = zeros_f
                    for d in range(D):
                        vd = plsc.load_gather(rows_v[b], [ridx, cols[d]])
                        diff = u[d] - vd
                        acc_sq = acc_sq + diff * diff
                        acc_nv = acc_nv + vd * vd
                    plsc.store_scatter(onv_v[b], [ridx], acc_nv)
                    plsc.store_scatter(osq_v[b], [ridx], acc_sq)
                    return 0

                lax.fori_loop(1, S, s_body, 0)

        def start_flush(k, b):
            base_e = (wid * bpw + k * C) * S
            return (pltpu.async_copy(onv_v[b],
                                     nv_hbm.at[pl.ds(base_e, rows_per_chunk)],
                                     osem[b]),
                    pltpu.async_copy(osq_v[b],
                                     sq_hbm.at[pl.ds(base_e, rows_per_chunk)],
                                     osem[b]))

        gather = {0: start_fetch(0, 0)}
        flush = {}
        for k in range(nchunk):
            b = k % 2
            if k + 1 < nchunk:
                gather[k + 1] = start_fetch(k + 1, (k + 1) % 2)
            gather.pop(k).wait()
            if k - 2 in flush:
                for h in flush.pop(k - 2):
                    h.wait()
            compute(b)
            flush[k] = start_flush(k, b)
        for hs in flush.values():
            for h in hs:
                h.wait()

    mesh = plsc.VectorSubcoreMesh(core_axis_name="c", subcore_axis_name="s")
    f = pl.kernel(
        body,
        out_type=[jax.ShapeDtypeStruct((B * S,), jnp.float32),
                  jax.ShapeDtypeStruct((B * S,), jnp.float32)],
        mesh=mesh,
        compiler_params=pltpu.CompilerParams(needs_layout_passes=False,
                                             use_tc_tiling_on_sc=False),
        scratch_types=[
            pltpu.VMEM((rows_per_chunk,), jnp.int32),
            pltpu.VMEM((rows_per_chunk,), jnp.int32),
            pltpu.VMEM((rows_per_chunk, D), jnp.float32),
            pltpu.VMEM((rows_per_chunk, D), jnp.float32),
            pltpu.VMEM((rows_per_chunk,), jnp.float32),
            pltpu.VMEM((rows_per_chunk,), jnp.float32),
            pltpu.VMEM((rows_per_chunk,), jnp.float32),
            pltpu.VMEM((rows_per_chunk,), jnp.float32),
            pltpu.SemaphoreType.DMA,
            pltpu.SemaphoreType.DMA,
            pltpu.SemaphoreType.DMA,
            pltpu.SemaphoreType.DMA,
        ],
    )
    return f(weight, idx_flat)


def _tc_transpose_body(wt_ref, out_ref):
    y = wt_ref[...].T
    y4 = y.reshape(y.shape[0] // 64, 8, 8, y.shape[1])
    pieces = [y4[:, m].reshape(y.shape[0] // 8, y.shape[1]) for m in range(8)]
    out_ref[...] = jnp.concatenate(pieces, axis=1)


def _tc_transpose(wt, N, D):
    CT = 4096
    R = N * D // 128
    RB = CT * D // 128
    return pl.pallas_call(
        _tc_transpose_body,
        grid=(pl.cdiv(N, CT),),
        in_specs=[pl.BlockSpec((D, CT), lambda i: (0, i))],
        out_specs=pl.BlockSpec((RB, 128), lambda i: (i, 0)),
        out_shape=jax.ShapeDtypeStruct((R, 128), jnp.float32),
    )(wt)


def _tc_finish_body(nv_ref, sq_ref, out_ref):
    nv = nv_ref[...]
    sq = sq_ref[...]
    nu = nv[:, 0:1]
    su = jnp.where(nu > 1.0, 1.0 / (jnp.sqrt(nu) + 1e-7), 1.0)
    sv = jnp.where(nv > 1.0, 1.0 / (jnp.sqrt(nv) + 1e-7), 1.0)
    nu_c = jnp.clip(su * su * nu, 0.0, 1.0 - EPS)
    nv_c = jnp.clip(sv * sv * nv, 0.0, 1.0 - EPS)
    sq_s = su * su * nu + sv * sv * nv - su * sv * (nu + nv - sq)
    x = 1.0 + 2.0 * sq_s / ((1.0 - nu_c) * (1.0 - nv_c))
    x = jnp.maximum(x, 1.0 + EPS)
    dist = jnp.log(x + jnp.sqrt((x - 1.0) * (x + 1.0)))
    out_ref[...] = dist[:, 1:]


def _tc_finish(nv, sq, B, S):
    blk = 512
    return pl.pallas_call(
        _tc_finish_body,
        grid=(B // blk,),
        in_specs=[pl.BlockSpec((blk, S), lambda i: (i, 0)),
                  pl.BlockSpec((blk, S), lambda i: (i, 0))],
        out_specs=pl.BlockSpec((blk, S - 1), lambda i: (i, 0)),
        out_shape=jax.ShapeDtypeStruct((B, S - 1), jnp.float32),
    )(nv, sq)


@jax.jit
def kernel(inputs, weight):
    B, S = inputs.shape
    N, D = weight.shape
    w_lin = _tc_transpose(weight.T, N, D)
    idx = inputs.reshape(B * S)
    idx_flat = (idx & ~63) | ((idx & 7) << 3) | ((idx >> 3) & 7)
    nv_flat, sq_flat = _sc_gather_reduce(w_lin.reshape(N, D), idx_flat,
                                         B, S, D)
    nv = nv_flat.reshape(B, S)
    sq = sq_flat.reshape(B, S)
    return _tc_finish(nv, sq, B, S)


# speedup vs baseline: 2.6089x; 1.0278x over previous
import functools

import jax
import jax.numpy as jnp
from jax import lax
from jax.experimental import pallas as pl
from jax.experimental.pallas import tpu as pltpu
from jax.experimental.pallas import tpu_sc as plsc

EPS = 1e-05
L = 16
NC = 2
NS = 16
NW = NC * NS


def _sc_gather_reduce(weight, idx_flat, B, S, D):
    bpw = B // NW
    C = 32
    nchunk = bpw // C
    rows_per_chunk = C * S

    def body(weight_hbm, idx_hbm, nv_hbm, sq_hbm,
             idx_v0, idx_v1, rows_v0, rows_v1, onv_v0, onv_v1,
             osq_v0, osq_v1, gsem0, gsem1, osem0, osem1):
        idx_v = (idx_v0, idx_v1)
        rows_v = (rows_v0, rows_v1)
        onv_v = (onv_v0, onv_v1)
        osq_v = (osq_v0, osq_v1)
        gsem = (gsem0, gsem1)
        osem = (osem0, osem1)
        wid = lax.axis_index("s") * NC + lax.axis_index("c")
        iota = lax.iota(jnp.int32, L)
        zeros_f = jnp.zeros((L,), jnp.float32)
        cols = [(jnp.full((L,), d, jnp.int32) + iota) & (D - 1)
                for d in range(D)]
        rbases = [(g * L + iota) * S for g in range(C // L)]

        def start_fetch(k, b):
            base_e = (wid * bpw + k * C) * S
            pltpu.sync_copy(idx_hbm.at[pl.ds(base_e, rows_per_chunk)],
                            idx_v[b])
            return pltpu.async_copy(weight_hbm.at[idx_v[b]],
                                    rows_v[b], gsem[b])

        def compute(b):
            for g in range(C // L):
                rbase = rbases[g]
                u = [plsc.load_gather(rows_v[b], [rbase, cols[d]])
                     for d in range(D)]
                nu = u[0] * u[0]
                for d in range(1, D):
                    nu = nu + u[d] * u[d]
                plsc.store_scatter(onv_v[b], [rbase], nu)
                plsc.store_scatter(osq_v[b], [rbase], zeros_f)

                def s_body(s, _):
                    ridx = rbase + s
                    acc_sq = zeros_f
                    acc_nv = zeros_f
                    for d in range(D):
                        vd = plsc.load_gather(rows_v[b], [ridx, cols[d]])
                        diff = u[d] - vd
                        acc_sq = acc_sq + diff * diff
                        acc_nv = acc_nv + vd * vd
                    plsc.store_scatter(onv_v[b], [ridx], acc_nv)
                    plsc.store_scatter(osq_v[b], [ridx], acc_sq)
                    return 0

                lax.fori_loop(1, S, s_body, 0)

        def start_flush(k, b):
            base_e = (wid * bpw + k * C) * S
            return (pltpu.async_copy(onv_v[b],
                                     nv_hbm.at[pl.ds(base_e, rows_per_chunk)],
                                     osem[b]),
                    pltpu.async_copy(osq_v[b],
                                     sq_hbm.at[pl.ds(base_e, rows_per_chunk)],
                                     osem[b]))

        gather = {0: start_fetch(0, 0)}
        flush = {}
        for k in range(nchunk):
            b = k % 2
            if k + 1 < nchunk:
                gather[k + 1] = start_fetch(k + 1, (k + 1) % 2)
            gather.pop(k).wait()
            if k - 2 in flush:
                for h in flush.pop(k - 2):
                    h.wait()
            compute(b)
            flush[k] = start_flush(k, b)
        for hs in flush.values():
            for h in hs:
                h.wait()

    mesh = plsc.VectorSubcoreMesh(core_axis_name="c", subcore_axis_name="s")
    f = pl.kernel(
        body,
        out_type=[jax.ShapeDtypeStruct((B * S,), jnp.float32),
                  jax.ShapeDtypeStruct((B * S,), jnp.float32)],
        mesh=mesh,
        compiler_params=pltpu.CompilerParams(needs_layout_passes=False,
                                             use_tc_tiling_on_sc=False),
        scratch_types=[
            pltpu.VMEM((rows_per_chunk,), jnp.int32),
            pltpu.VMEM((rows_per_chunk,), jnp.int32),
            pltpu.VMEM((rows_per_chunk, D), jnp.float32),
            pltpu.VMEM((rows_per_chunk, D), jnp.float32),
            pltpu.VMEM((rows_per_chunk,), jnp.float32),
            pltpu.VMEM((rows_per_chunk,), jnp.float32),
            pltpu.VMEM((rows_per_chunk,), jnp.float32),
            pltpu.VMEM((rows_per_chunk,), jnp.float32),
            pltpu.SemaphoreType.DMA,
            pltpu.SemaphoreType.DMA,
            pltpu.SemaphoreType.DMA,
            pltpu.SemaphoreType.DMA,
        ],
    )
    return f(weight, idx_flat)


def _tc_transpose_body(wt_ref, out_ref):
    y = wt_ref[...].T
    y4 = y.reshape(y.shape[0] // 64, 8, 8, y.shape[1])
    pieces = [y4[:, m].reshape(y.shape[0] // 8, y.shape[1]) for m in range(8)]
    out_ref[...] = jnp.concatenate(pieces, axis=1)


def _tc_transpose(wt, N, D):
    CT = 8192
    R = N * D // 128
    RB = CT * D // 128
    return pl.pallas_call(
        _tc_transpose_body,
        grid=(pl.cdiv(N, CT),),
        in_specs=[pl.BlockSpec((D, CT), lambda i: (0, i))],
        out_specs=pl.BlockSpec((RB, 128), lambda i: (i, 0)),
        out_shape=jax.ShapeDtypeStruct((R, 128), jnp.float32),
    )(wt)


def _tc_finish_body(nv_ref, sq_ref, out_ref):
    nv = nv_ref[...]
    sq = sq_ref[...]
    nu = nv[:, 0:1]
    su = jnp.where(nu > 1.0, 1.0 / (jnp.sqrt(nu) + 1e-7), 1.0)
    sv = jnp.where(nv > 1.0, 1.0 / (jnp.sqrt(nv) + 1e-7), 1.0)
    nu_c = jnp.clip(su * su * nu, 0.0, 1.0 - EPS)
    nv_c = jnp.clip(sv * sv * nv, 0.0, 1.0 - EPS)
    sq_s = su * su * nu + sv * sv * nv - su * sv * (nu + nv - sq)
    x = 1.0 + 2.0 * sq_s / ((1.0 - nu_c) * (1.0 - nv_c))
    x = jnp.maximum(x, 1.0 + EPS)
    dist = jnp.log(x + jnp.sqrt((x - 1.0) * (x + 1.0)))
    out_ref[...] = dist[:, 1:]


def _tc_finish(nv, sq, B, S):
    blk = 512
    return pl.pallas_call(
        _tc_finish_body,
        grid=(B // blk,),
        in_specs=[pl.BlockSpec((blk, S), lambda i: (i, 0)),
                  pl.BlockSpec((blk, S), lambda i: (i, 0))],
        out_specs=pl.BlockSpec((blk, S - 1), lambda i: (i, 0)),
        out_shape=jax.ShapeDtypeStruct((B, S - 1), jnp.float32),
    )(nv, sq)


@jax.jit
def kernel(inputs, weight):
    B, S = inputs.shape
    N, D = weight.shape
    w_lin = _tc_transpose(weight.T, N, D)
    idx = inputs.reshape(B * S)
    idx_flat = (idx & ~63) | ((idx & 7) << 3) | ((idx >> 3) & 7)
    nv_flat, sq_flat = _sc_gather_reduce(w_lin.reshape(N, D), idx_flat,
                                         B, S, D)
    nv = nv_flat.reshape(B, S)
    sq = sq_flat.reshape(B, S)
    return _tc_finish(nv, sq, B, S)
